# baseline (speedup 1.0000x reference)
import functools

import jax
import jax.numpy as jnp
from jax import lax
from jax.experimental import pallas as pl
from jax.experimental.pallas import tpu as pltpu
from jax.experimental.pallas import tpu_sc as plsc
from jax._src.pallas import mpmd as _pl_mpmd

TOPK = 64
MAX_ALPHA = 3.0
NEG = -1000000000.0


def _q_body(qv_ref, wq_ref, q_ref):
    q_ref[...] = lax.dot_general(
        qv_ref[...], wq_ref[...],
        dimension_numbers=(((1,), (1,)), ((), ())),
        preferred_element_type=jnp.float32,
    )


def _p1_body(q_ref, dec_ref, z_ref, scores_ref):
    s = jnp.dot(q_ref[...], dec_ref[...], preferred_element_type=jnp.float32)
    colmax = jnp.max(z_ref[...], axis=0)
    scores_ref[...] = s + jnp.where(colmax > 0.0, 0.0, NEG)[None, :]


def _p2_body(scores_ref, ls_ref, bvec_ref):
    R = scores_ref.shape[0]
    s = scores_ref[...]
    b = lax.bitcast_convert_type(s, jnp.int32)
    key = jnp.where(b >= 0, b, b ^ jnp.int32(0x7FFFFFFF))

    def count_ge(t):
        return jnp.sum(jnp.where(key >= t, 1.0, 0.0))

    imin = jnp.int32(-(2**31))
    imax = jnp.int32(2**31 - 1)
    k = jnp.float32(TOPK)
    pos_ok = count_ge(jnp.int32(0)) >= k
    lo = jnp.where(pos_ok, jnp.int32(0), imin)
    hi = jnp.where(pos_ok, imax, jnp.int32(-1))

    def bs_body(_, carry):
        lo, hi = carry
        d = hi - lo
        m1 = lo + jnp.maximum(jnp.int32(1), d >> 2)
        m2 = lo + (d >> 1) + (d & 1)
        m3 = m2 + (d >> 2)
        ok1 = count_ge(m1) >= k
        ok2 = count_ge(m2) >= k
        ok3 = count_ge(m3) >= k
        nlo = jnp.where(ok3, m3, jnp.where(ok2, m2, jnp.where(ok1, m1, lo)))
        nhi = jnp.where(ok3, hi, jnp.where(ok2, m3 - 1,
                        jnp.where(ok1, m2 - 1, m1 - 1)))
        return nlo, nhi

    carry = (lo, hi)
    for _i in range(18):
        carry = bs_body(_i, carry)
    lo, hi = carry
    thr = lo

    sel = jnp.where(key > thr, 1.0, 0.0)
    ties = jnp.where(key == thr, 1.0, 0.0)
    m = TOPK - jnp.sum(sel).astype(jnp.int32)
    flat = (lax.broadcasted_iota(jnp.int32, (R, 128), 0) * 128
            + lax.broadcasted_iota(jnp.int32, (R, 128), 1))
    big = jnp.int32(2**30)

    def tie_body(_, carry):
        sel, ties = carry
        idx = jnp.min(jnp.where(ties > 0.0, flat, big))
        hit = jnp.where(flat == idx, 1.0, 0.0)
        return jnp.maximum(sel, hit), ties * (1.0 - hit)

    sel, _ = lax.fori_loop(0, m, tie_body, (sel, ties))

    scale = jnp.minimum(jnp.exp(ls_ref[0]), 10.0)
    boost = 1.0 + (MAX_ALPHA - 1.0) / (1.0 + jnp.exp(-s * scale))
    bvec_ref[...] = jnp.where(sel > 0.0, boost, 1.0)


def _sc_ones(T, L):
    NW = 32
    rows_per = T // NW
    mesh = plsc.VectorSubcoreMesh(core_axis_name="c", subcore_axis_name="s")

    @functools.partial(
        pl.kernel,
        out_type=jax.ShapeDtypeStruct((T, L), jnp.float32),
        mesh=mesh,
        scratch_types=[
            pltpu.VMEM((2, L), jnp.float32),
            pltpu.SemaphoreType.DMA,
        ],
    )
    def k(out_hbm, buf, sem):
        wid = lax.axis_index("s") * 2 + lax.axis_index("c")
        r0 = wid * rows_per

        def fill(i, carry):
            buf[0, pl.ds(i * 16, 16)] = jnp.ones((16,), jnp.float32)
            buf[1, pl.ds(i * 16, 16)] = jnp.ones((16,), jnp.float32)
            return carry

        lax.fori_loop(0, L // 16, fill, 0, unroll=8)
        for i in range(rows_per // 2):
            pltpu.sync_copy(buf, out_hbm.at[pl.ds(r0 + 2 * i, 2), :])

    return k


def _sc_patch(T, L):
    NW = 32
    CW = L // NW
    mesh = plsc.VectorSubcoreMesh(core_axis_name="c", subcore_axis_name="s")

    def body(bvec_hbm, z_hbm, ones_hbm, out_hbm, bv, colbuf, pbuf, sem):
        del ones_hbm, sem
        wid = lax.axis_index("s") * 2 + lax.axis_index("c")
        c0 = wid * CW
        pltpu.sync_copy(bvec_hbm.at[pl.ds(c0, CW)], bv)
        lane_iota = lax.iota(jnp.int32, 16)

        def outer(v, carry):
            vec = bv[pl.ds(v * 16, 16)]
            mask0 = jnp.where(vec != 1.0, jnp.int32(1), jnp.int32(0))
            anyset = lax.reduce_max_p.bind(mask0, axes=(0,))

            @pl.when(anyset > 0)
            def _():
                def cond(m):
                    return lax.reduce_max_p.bind(m, axes=(0,)) > 0

                def wbody(m):
                    lane = lax.reduce_min_p.bind(
                        jnp.where(m > 0, lane_iota, jnp.int32(16)), axes=(0,))
                    c = c0 + v * 16 + lane
                    boost = lax.reduce_max_p.bind(
                        jnp.where(lane_iota == lane, vec, jnp.float32(-3e38)),
                        axes=(0,))
                    pltpu.sync_copy(z_hbm.at[:, c], colbuf)

                    def inner(i, acc):
                        zv = colbuf[pl.ds(i * 16, 16)]
                        pbuf[pl.ds(i * 16, 16)] = jnp.where(
                            zv > 0.0, boost, 1.0)
                        return acc

                    lax.fori_loop(0, T // 16, inner, 0)
                    pltpu.sync_copy(pbuf, out_hbm.at[:, c])
                    return m * jnp.where(lane_iota == lane, jnp.int32(0),
                                         jnp.int32(1))

                lax.while_loop(cond, wbody, mask0)

            return carry

        lax.fori_loop(0, CW // 16, outer, 0)

    return _pl_mpmd._mpmd_map(
        [(mesh, body)],
        jax.ShapeDtypeStruct((T, L), jnp.float32),
        input_output_aliases={2: 0},
        scratch_types=[
            pltpu.VMEM((CW,), jnp.float32),
            pltpu.VMEM((T,), jnp.float32),
            pltpu.VMEM((T,), jnp.float32),
            pltpu.SemaphoreType.DMA,
        ],
    )


def _p3_body(z_ref, bvec_ref, out_ref):
    out_ref[...] = jnp.where(z_ref[...] > 0.0, bvec_ref[...], 1.0)


def _ones_body(out_ref):
    out_ref[...] = jnp.ones_like(out_ref)


def kernel(question_vec, z, decoder_weight, W_q, log_scale):
    qv = question_vec.reshape(1, -1).astype(jnp.float32)
    T, L = z.shape
    H = W_q.shape[0]
    TL = 2048
    nblk = L // TL

    q = pl.pallas_call(
        _q_body,
        in_specs=[
            pl.BlockSpec((1, H), lambda: (0, 0)),
            pl.BlockSpec((H, H), lambda: (0, 0)),
        ],
        out_specs=pl.BlockSpec((1, H), lambda: (0, 0)),
        out_shape=jax.ShapeDtypeStruct((1, H), jnp.float32),
    )(qv, W_q)

    scores = pl.pallas_call(
        _p1_body,
        grid=(nblk,),
        in_specs=[
            pl.BlockSpec((1, H), lambda i: (0, 0)),
            pl.BlockSpec((H, TL), lambda i: (0, i)),
            pl.BlockSpec((T, TL), lambda i: (0, i)),
        ],
        out_specs=pl.BlockSpec((1, TL), lambda i: (0, i)),
        out_shape=jax.ShapeDtypeStruct((1, L), jnp.float32),
    )(q, decoder_weight, z)

    R = L // 128
    bvec = pl.pallas_call(
        _p2_body,
        in_specs=[
            pl.BlockSpec((R, 128), lambda: (0, 0)),
            pl.BlockSpec(memory_space=pltpu.SMEM),
        ],
        out_specs=pl.BlockSpec((R, 128), lambda: (0, 0)),
        out_shape=jax.ShapeDtypeStruct((R, 128), jnp.float32),
    )(scores.reshape(R, 128), log_scale)

    out = pl.pallas_call(
        _p3_body,
        grid=(nblk,),
        in_specs=[
            pl.BlockSpec((T, TL), lambda i: (0, i)),
            pl.BlockSpec((1, TL), lambda i: (0, i)),
        ],
        out_specs=pl.BlockSpec((T, TL), lambda i: (0, i)),
        out_shape=jax.ShapeDtypeStruct((T, L), z.dtype),
    )(z, bvec.reshape(1, L))

    return out


# speedup vs baseline: 1.0352x; 1.0352x over previous
import jax
import jax.numpy as jnp
from jax import lax
from jax.experimental import pallas as pl
from jax.experimental.pallas import tpu as pltpu

TOPK = 64
MAX_ALPHA = 3.0
NEG = -1000000000.0


def _p1_body(qv_ref, wq_ref, dec_ref, z_ref, scores_ref, q_scr):
    i = pl.program_id(0)

    @pl.when(i == 0)
    def _():
        q_scr[...] = lax.dot_general(
            qv_ref[...], wq_ref[...],
            dimension_numbers=(((1,), (1,)), ((), ())),
            preferred_element_type=jnp.float32,
        )

    s = jnp.dot(q_scr[...], dec_ref[...], preferred_element_type=jnp.float32)
    colmax = jnp.max(z_ref[...], axis=0)
    scores_ref[...] = s + jnp.where(colmax > 0.0, 0.0, NEG)[None, :]


def _p2_body(scores_ref, ls_ref, bvec_ref):
    R = scores_ref.shape[0]
    s = scores_ref[...]
    b = lax.bitcast_convert_type(s, jnp.int32)
    key = jnp.where(b >= 0, b, b ^ jnp.int32(0x7FFFFFFF))

    def count_ge(t):
        return jnp.sum(jnp.where(key >= t, 1.0, 0.0))

    imin = jnp.int32(-(2**31))
    imax = jnp.int32(2**31 - 1)
    k = jnp.float32(TOPK)
    pos_ok = count_ge(jnp.int32(0)) >= k
    lo = jnp.where(pos_ok, jnp.int32(0), imin)
    hi = jnp.where(pos_ok, imax, jnp.int32(-1))

    def bs_body(carry):
        lo, hi = carry
        d = hi - lo
        m1 = lo + jnp.maximum(jnp.int32(1), d >> 2)
        m2 = lo + (d >> 1) + (d & 1)
        m3 = m2 + (d >> 2)
        ok1 = count_ge(m1) >= k
        ok2 = count_ge(m2) >= k
        ok3 = count_ge(m3) >= k
        nlo = jnp.where(ok3, m3, jnp.where(ok2, m2, jnp.where(ok1, m1, lo)))
        nhi = jnp.where(ok3, hi, jnp.where(ok2, m3 - 1,
                        jnp.where(ok1, m2 - 1, m1 - 1)))
        return nlo, nhi

    carry = (lo, hi)
    for _ in range(18):
        carry = bs_body(carry)
    lo, hi = carry
    thr = lo

    sel = jnp.where(key > thr, 1.0, 0.0)
    ties = jnp.where(key == thr, 1.0, 0.0)
    m = TOPK - jnp.sum(sel).astype(jnp.int32)
    flat = (lax.broadcasted_iota(jnp.int32, (R, 128), 0) * 128
            + lax.broadcasted_iota(jnp.int32, (R, 128), 1))
    big = jnp.int32(2**30)

    def tie_body(_, carry):
        sel, ties = carry
        idx = jnp.min(jnp.where(ties > 0.0, flat, big))
        hit = jnp.where(flat == idx, 1.0, 0.0)
        return jnp.maximum(sel, hit), ties * (1.0 - hit)

    sel, _ = lax.fori_loop(0, m, tie_body, (sel, ties))

    scale = jnp.minimum(jnp.exp(ls_ref[0]), 10.0)
    boost = 1.0 + (MAX_ALPHA - 1.0) / (1.0 + jnp.exp(-s * scale))
    bvec_ref[...] = jnp.where(sel > 0.0, boost, 1.0)


def _p3_body(z_ref, bvec_ref, out_ref):
    out_ref[...] = jnp.where(z_ref[...] > 0.0, bvec_ref[...], 1.0)


def kernel(question_vec, z, decoder_weight, W_q, log_scale):
    qv = question_vec.reshape(1, -1).astype(jnp.float32)
    T, L = z.shape
    H = W_q.shape[0]
    TL = 1024
    nblk = L // TL

    scores = pl.pallas_call(
        _p1_body,
        grid=(nblk,),
        in_specs=[
            pl.BlockSpec((1, H), lambda i: (0, 0)),
            pl.BlockSpec((H, H), lambda i: (0, 0)),
            pl.BlockSpec((H, TL), lambda i: (0, i)),
            pl.BlockSpec((T, TL), lambda i: (0, i)),
        ],
        out_specs=pl.BlockSpec((1, TL), lambda i: (0, i)),
        out_shape=jax.ShapeDtypeStruct((1, L), jnp.float32),
        scratch_shapes=[pltpu.VMEM((1, H), jnp.float32)],
    )(qv, W_q, decoder_weight, z)

    R = L // 128
    bvec = pl.pallas_call(
        _p2_body,
        in_specs=[
            pl.BlockSpec((R, 128), lambda: (0, 0)),
            pl.BlockSpec(memory_space=pltpu.SMEM),
        ],
        out_specs=pl.BlockSpec((R, 128), lambda: (0, 0)),
        out_shape=jax.ShapeDtypeStruct((R, 128), jnp.float32),
    )(scores.reshape(R, 128), log_scale)

    TO = 2048
    out = pl.pallas_call(
        _p3_body,
        grid=(L // TO,),
        in_specs=[
            pl.BlockSpec((T, TO), lambda i: (0, i)),
            pl.BlockSpec((1, TO), lambda i: (0, i)),
        ],
        out_specs=pl.BlockSpec((T, TO), lambda i: (0, i)),
        out_shape=jax.ShapeDtypeStruct((T, L), z.dtype),
    )(z, bvec.reshape(1, L))

    return out
